# baseline (speedup 1.0000x reference)
import jax
import jax.numpy as jnp
from jax.experimental import pallas as pl
from jax.experimental.pallas import tpu as pltpu

_PNUM = 1024
_RB = 8
_NO = _PNUM // 128
_NQ = 128 // _RB


def _poly_loss_kernel(p_ref, g_ref, o_ref, s_ref):
    px = p_ref[0, 0:1, :]
    py = p_ref[0, 1:2, :]
    gx = g_ref[0, 0:1, :]
    gy = g_ref[0, 1:2, :]

    def _roll(v, r):
        return v if r == 0 else jnp.roll(v, -r, axis=1)

    gxc = jnp.concatenate([_roll(gx, r) for r in range(_RB)], axis=0)
    gyc = jnp.concatenate([_roll(gy, r) for r in range(_RB)], axis=0)

    pxb = jnp.broadcast_to(px, (_RB, _PNUM))
    pyb = jnp.broadcast_to(py, (_RB, _PNUM))
    pxo = [pxb] + [jnp.roll(pxb, 128 * o, axis=1) for o in range(1, _NO)]
    pyo = [pyb] + [jnp.roll(pyb, 128 * o, axis=1) for o in range(1, _NO)]

    def smooth2(d):
        a = jnp.abs(d)
        m = jnp.minimum(a, 1.0)
        return m * (a + a - m)

    for u in range(_NQ):
        for o in range(_NO):
            f = smooth2(pxo[o] - gxc) + smooth2(pyo[o] - gyc)
            blk = u * _NO + o
            s_ref[_RB * blk : _RB * (blk + 1), :] = f
        if u + 1 < _NQ:
            gxc = jnp.roll(gxc, -_RB, axis=1)
            gyc = jnp.roll(gyc, -_RB, axis=1)

    ones_m = jnp.ones((_PNUM, 8), dtype=jnp.float32)
    dis = jax.lax.dot_general(
        s_ref[:, :], ones_m, (((1,), (0,)), ((), ())),
        preferred_element_type=jnp.float32,
    )
    b = pl.num_programs(0)
    step = pl.program_id(0)
    part = jnp.min(dis, axis=(0, 1), keepdims=True)

    @pl.when(step == 0)
    def _init():
        o_ref[0, :, :] = part

    @pl.when(step > 0)
    def _accum():
        o_ref[0, :, :] = o_ref[0, :, :] + part

    @pl.when(step == b - 1)
    def _scale():
        o_ref[0, :, :] = o_ref[0, :, :] * (1.0 / (2.0 * _PNUM * b))


@jax.jit
def kernel(pred, gt):
    b = pred.shape[0]
    p = jnp.transpose(pred, (0, 2, 1))
    g = jnp.transpose(gt, (0, 2, 1))
    mins = pl.pallas_call(
        _poly_loss_kernel,
        grid=(b,),
        in_specs=[
            pl.BlockSpec((1, 2, _PNUM), lambda i: (i, 0, 0)),
            pl.BlockSpec((1, 2, _PNUM), lambda i: (i, 0, 0)),
        ],
        out_specs=pl.BlockSpec((1, 1, 1), lambda i: (0, 0, 0)),
        out_shape=jax.ShapeDtypeStruct((1, 1, 1), jnp.float32),
        scratch_shapes=[pltpu.VMEM((_PNUM, _PNUM), jnp.float32)],
        compiler_params=pltpu.CompilerParams(
            dimension_semantics=("arbitrary",),
        ),
    )(p, g)
    return mins[0, 0, 0]


# speedup vs baseline: 1.0066x; 1.0066x over previous
import jax
import jax.numpy as jnp
from jax.experimental import pallas as pl
from jax.experimental.pallas import tpu as pltpu

_PNUM = 1024
_RB = 8
_NO = _PNUM // 128
_NQ = 128 // _RB


def _poly_loss_kernel(p_ref, g_ref, o_ref, s_ref):
    px = p_ref[0, 0:1, :]
    py = p_ref[0, 1:2, :]
    gx = g_ref[0, 0:1, :]
    gy = g_ref[0, 1:2, :]

    def _roll(v, r):
        return v if r == 0 else jnp.roll(v, -r, axis=1)

    gxc = jnp.concatenate([_roll(gx, r) for r in range(_RB)], axis=0)
    gyc = jnp.concatenate([_roll(gy, r) for r in range(_RB)], axis=0)

    pxb = jnp.broadcast_to(px, (_RB, _PNUM))
    pyb = jnp.broadcast_to(py, (_RB, _PNUM))
    pxo = [pxb] + [jnp.roll(pxb, 128 * o, axis=1) for o in range(1, _NO)]
    pyo = [pyb] + [jnp.roll(pyb, 128 * o, axis=1) for o in range(1, _NO)]

    def smooth2(d):
        a = jnp.abs(d)
        m = jnp.minimum(a, 1.0)
        return m * (a + a - m)

    for u in range(_NQ):
        for o in range(_NO):
            f = smooth2(pxo[o] - gxc) + smooth2(pyo[o] - gyc)
            blk = u * _NO + o
            s_ref[_RB * blk : _RB * (blk + 1), :] = f
        if u + 1 < _NQ:
            gxc = jnp.roll(gxc, -_RB, axis=1)
            gyc = jnp.roll(gyc, -_RB, axis=1)

    ones_m = jnp.ones((_PNUM, 128), dtype=jnp.float32)
    dis = jax.lax.dot_general(
        s_ref[:, :], ones_m, (((1,), (0,)), ((), ())),
        preferred_element_type=jnp.float32,
    )
    b = pl.num_programs(0)
    step = pl.program_id(0)
    part = jnp.min(dis, axis=(0, 1), keepdims=True)

    @pl.when(step == 0)
    def _init():
        o_ref[0, :, :] = part

    @pl.when(step > 0)
    def _accum():
        o_ref[0, :, :] = o_ref[0, :, :] + part

    @pl.when(step == b - 1)
    def _scale():
        o_ref[0, :, :] = o_ref[0, :, :] * (1.0 / (2.0 * _PNUM * b))


@jax.jit
def kernel(pred, gt):
    b = pred.shape[0]
    p = jnp.transpose(pred, (0, 2, 1))
    g = jnp.transpose(gt, (0, 2, 1))
    mins = pl.pallas_call(
        _poly_loss_kernel,
        grid=(b,),
        in_specs=[
            pl.BlockSpec((1, 2, _PNUM), lambda i: (i, 0, 0)),
            pl.BlockSpec((1, 2, _PNUM), lambda i: (i, 0, 0)),
        ],
        out_specs=pl.BlockSpec((1, 1, 1), lambda i: (0, 0, 0)),
        out_shape=jax.ShapeDtypeStruct((1, 1, 1), jnp.float32),
        scratch_shapes=[pltpu.VMEM((_PNUM, _PNUM), jnp.float32)],
        compiler_params=pltpu.CompilerParams(
            dimension_semantics=("arbitrary",),
        ),
    )(p, g)
    return mins[0, 0, 0]


# speedup vs baseline: 1.0134x; 1.0068x over previous
import jax
import jax.numpy as jnp
from jax.experimental import pallas as pl
from jax.experimental.pallas import tpu as pltpu

_PNUM = 1024
_RB = 8
_NO = _PNUM // 128
_NQ = 128 // _RB


def _poly_loss_kernel(p_ref, g_ref, o_ref, s_ref):
    px = p_ref[0, 0:1, :]
    py = p_ref[0, 1:2, :]
    gx = g_ref[0, 0:1, :]
    gy = g_ref[0, 1:2, :]

    def _roll(v, r):
        return v if r == 0 else jnp.roll(v, -r, axis=1)

    gxc = jnp.concatenate([_roll(gx, r) for r in range(_RB)], axis=0)
    gyc = jnp.concatenate([_roll(gy, r) for r in range(_RB)], axis=0)

    pxb = jnp.broadcast_to(px, (_RB, _PNUM))
    pyb = jnp.broadcast_to(py, (_RB, _PNUM))
    pxo = [pxb] + [jnp.roll(pxb, 128 * o, axis=1) for o in range(1, _NO)]
    pyo = [pyb] + [jnp.roll(pyb, 128 * o, axis=1) for o in range(1, _NO)]

    def smooth2(d):
        a = jnp.abs(d)
        m = jnp.minimum(a, 1.0)
        return m * (a + a - m)

    ones_m = jnp.ones((_PNUM, 128), dtype=jnp.float32)
    parts = []
    for u in range(_NQ):
        for o in range(_NO):
            f = smooth2(pxo[o] - gxc) + smooth2(pyo[o] - gyc)
            blk = u * _NO + o
            s_ref[_RB * blk : _RB * (blk + 1), :] = f
        if (u + 1) % 4 == 0:
            c = u // 4
            rows = _RB * _NO * 4
            dis = jax.lax.dot_general(
                s_ref[rows * c : rows * (c + 1), :], ones_m,
                (((1,), (0,)), ((), ())),
                preferred_element_type=jnp.float32,
            )
            parts.append(jnp.min(dis, axis=(0, 1), keepdims=True))
        if u + 1 < _NQ:
            gxc = jnp.roll(gxc, -_RB, axis=1)
            gyc = jnp.roll(gyc, -_RB, axis=1)

    b = pl.num_programs(0)
    step = pl.program_id(0)
    part = jnp.minimum(
        jnp.minimum(parts[0], parts[1]), jnp.minimum(parts[2], parts[3])
    )

    @pl.when(step == 0)
    def _init():
        o_ref[0, :, :] = part

    @pl.when(step > 0)
    def _accum():
        o_ref[0, :, :] = o_ref[0, :, :] + part

    @pl.when(step == b - 1)
    def _scale():
        o_ref[0, :, :] = o_ref[0, :, :] * (1.0 / (2.0 * _PNUM * b))


@jax.jit
def kernel(pred, gt):
    b = pred.shape[0]
    p = jnp.transpose(pred, (0, 2, 1))
    g = jnp.transpose(gt, (0, 2, 1))
    mins = pl.pallas_call(
        _poly_loss_kernel,
        grid=(b,),
        in_specs=[
            pl.BlockSpec((1, 2, _PNUM), lambda i: (i, 0, 0)),
            pl.BlockSpec((1, 2, _PNUM), lambda i: (i, 0, 0)),
        ],
        out_specs=pl.BlockSpec((1, 1, 1), lambda i: (0, 0, 0)),
        out_shape=jax.ShapeDtypeStruct((1, 1, 1), jnp.float32),
        scratch_shapes=[pltpu.VMEM((_PNUM, _PNUM), jnp.float32)],
        compiler_params=pltpu.CompilerParams(
            dimension_semantics=("arbitrary",),
        ),
    )(p, g)
    return mins[0, 0, 0]
